# baseline (speedup 1.0000x reference)
import functools

import jax
import jax.numpy as jnp
from jax import lax
from jax.experimental import pallas as pl
from jax.experimental.pallas import tpu as pltpu
from jax.experimental.pallas import tpu_sc as plsc

_B, _S, _V, _D, _NDOM, _L = 1024, 50, 100000, 768, 6, 32

_NC, _NS, _LANES = 2, 16, 16
_NW = _NC * _NS
_BPW = _B // _NW

_mesh = plsc.VectorSubcoreMesh(core_axis_name="c", subcore_axis_name="s")


@functools.partial(
    pl.kernel,
    mesh=_mesh,
    out_type=[
        jax.ShapeDtypeStruct((_B, _D), jnp.float32),
        jax.ShapeDtypeStruct((_B, _D), jnp.float32),
    ],
    scratch_types=[
        pltpu.VMEM((_BPW,), jnp.int32),
        pltpu.VMEM((_BPW,), jnp.int32),
        pltpu.VMEM((_BPW,), jnp.int32),
        pltpu.VMEM((_BPW,), jnp.int32),
        pltpu.VMEM((_BPW,), jnp.int32),
        pltpu.VMEM((_BPW,), jnp.int32),
        pltpu.VMEM((_BPW, _D), jnp.float32),
        pltpu.VMEM((_BPW, _D), jnp.float32),
        pltpu.SemaphoreType.DMA,
        pltpu.SemaphoreType.DMA,
    ],
)
def _sc_gather(sent_hbm, e1_hbm, e2_hbm, table_hbm, out1_hbm, out2_hbm,
               e1_v, e2_v, flat1_v, flat2_v, tok1_v, tok2_v,
               rows1_v, rows2_v, sem1, sem2):
    wid = lax.axis_index("s") * _NC + lax.axis_index("c")
    base = wid * _BPW
    pltpu.sync_copy(e1_hbm.at[pl.ds(base, _BPW)], e1_v)
    pltpu.sync_copy(e2_hbm.at[pl.ds(base, _BPW)], e2_v)
    for k in range(_BPW // _LANES):
        sl = pl.ds(k * _LANES, _LANES)
        row_base = (lax.iota(jnp.int32, _LANES) + (base + k * _LANES)) * _S
        flat1_v[sl] = row_base + e1_v[sl]
        flat2_v[sl] = row_base + e2_v[sl]
    tk1 = pltpu.async_copy(sent_hbm.at[flat1_v], tok1_v, sem1)
    tk2 = pltpu.async_copy(sent_hbm.at[flat2_v], tok2_v, sem2)
    tk1.wait()
    cp1 = pltpu.async_copy(table_hbm.at[tok1_v], rows1_v, sem1)
    tk2.wait()
    cp2 = pltpu.async_copy(table_hbm.at[tok2_v], rows2_v, sem2)
    cp1.wait()
    o1 = pltpu.async_copy(rows1_v, out1_hbm.at[pl.ds(base, _BPW)], sem1)
    cp2.wait()
    o2 = pltpu.async_copy(rows2_v, out2_hbm.at[pl.ds(base, _BPW)], sem2)
    o1.wait()
    o2.wait()


_BLK = 1024


def _cls_body(x1_ref, x2_ref, dom_ref, ds_ref, wt_ref, b_ref,
              logits_ref, labels_ref):
    dom = dom_ref[:].reshape(_BLK, 1)
    acc = jnp.zeros((_BLK, _D), jnp.float32)
    for j in range(_NDOM):
        row = ds_ref[j, :]
        acc = jnp.where(dom == j, row[None, :], acc)
    x = jnp.concatenate([x1_ref[:] + acc, x2_ref[:] + acc], axis=-1)
    logits_t = lax.dot_general(wt_ref[:], x, (((1,), (1,)), ((), ())))
    logits_t = logits_t + b_ref[:][:, None]
    logits_ref[:] = logits_t
    m = jnp.max(logits_t, axis=0)
    idx = lax.broadcasted_iota(jnp.int32, (_L, _BLK), 0)
    labels_ref[:] = jnp.min(jnp.where(logits_t == m[None, :], idx, _L), axis=0)


_classify = pl.pallas_call(
    _cls_body,
    grid=(_B // _BLK,),
    in_specs=[
        pl.BlockSpec((_BLK, _D), lambda i: (i, 0)),
        pl.BlockSpec((_BLK, _D), lambda i: (i, 0)),
        pl.BlockSpec((_BLK,), lambda i: (i,)),
        pl.BlockSpec((_NDOM, _D), lambda i: (0, 0)),
        pl.BlockSpec((_L, 2 * _D), lambda i: (0, 0)),
        pl.BlockSpec((_L,), lambda i: (0,)),
    ],
    out_specs=[
        pl.BlockSpec((_L, _BLK), lambda i: (0, i)),
        pl.BlockSpec((_BLK,), lambda i: (i,)),
    ],
    out_shape=[
        jax.ShapeDtypeStruct((_L, _B), jnp.float32),
        jax.ShapeDtypeStruct((_B,), jnp.int32),
    ],
)


def kernel(sentences, entities_1, entities_2, domains, emb_table,
           dataset_embeds, W, b):
    sent_flat = sentences.astype(jnp.int32).reshape(_B * _S)
    e1 = entities_1.astype(jnp.int32)
    e2 = entities_2.astype(jnp.int32)
    dom = domains.astype(jnp.int32)
    x1, x2 = _sc_gather(sent_flat, e1, e2, emb_table)
    logits_t, labels_1d = _classify(x1, x2, dom, dataset_embeds, W.T, b)
    return labels_1d[:, None], logits_t.T


# speedup vs baseline: 1.0074x; 1.0074x over previous
import functools

import jax
import jax.numpy as jnp
from jax import lax
from jax.experimental import pallas as pl
from jax.experimental.pallas import tpu as pltpu
from jax.experimental.pallas import tpu_sc as plsc

_B, _S, _V, _D, _NDOM, _L = 1024, 50, 100000, 768, 6, 32

_NC, _NS, _LANES = 2, 16, 16
_NW = _NC * _NS
_BPW = _B // _NW

_mesh = plsc.VectorSubcoreMesh(core_axis_name="c", subcore_axis_name="s")


@functools.partial(
    pl.kernel,
    mesh=_mesh,
    out_type=[
        jax.ShapeDtypeStruct((_B, _D), jnp.float32),
        jax.ShapeDtypeStruct((_B, _D), jnp.float32),
    ],
    scratch_types=[
        pltpu.VMEM((_BPW,), jnp.int32),
        pltpu.VMEM((_BPW,), jnp.int32),
        pltpu.VMEM((_BPW,), jnp.int32),
        pltpu.VMEM((_BPW,), jnp.int32),
        pltpu.VMEM((_BPW,), jnp.int32),
        pltpu.VMEM((_BPW,), jnp.int32),
        pltpu.VMEM((_BPW, _D), jnp.float32),
        pltpu.VMEM((_BPW, _D), jnp.float32),
        pltpu.SemaphoreType.DMA,
        pltpu.SemaphoreType.DMA,
    ],
)
def _sc_gather(sent_hbm, e1_hbm, e2_hbm, table_hbm, out1_hbm, out2_hbm,
               e1_v, e2_v, flat1_v, flat2_v, tok1_v, tok2_v,
               rows1_v, rows2_v, sem1, sem2):
    wid = lax.axis_index("s") * _NC + lax.axis_index("c")
    base = wid * _BPW
    pltpu.sync_copy(e1_hbm.at[pl.ds(base, _BPW)], e1_v)
    pltpu.sync_copy(e2_hbm.at[pl.ds(base, _BPW)], e2_v)
    for k in range(_BPW // _LANES):
        sl = pl.ds(k * _LANES, _LANES)
        row_base = (lax.iota(jnp.int32, _LANES) + (base + k * _LANES)) * _S
        flat1_v[sl] = row_base + e1_v[sl]
        flat2_v[sl] = row_base + e2_v[sl]
    tk1 = pltpu.async_copy(sent_hbm.at[flat1_v], tok1_v, sem1)
    tk2 = pltpu.async_copy(sent_hbm.at[flat2_v], tok2_v, sem2)
    tk1.wait()
    cp1 = pltpu.async_copy(table_hbm.at[tok1_v], rows1_v, sem1)
    tk2.wait()
    cp2 = pltpu.async_copy(table_hbm.at[tok2_v], rows2_v, sem2)
    cp1.wait()
    o1 = pltpu.async_copy(rows1_v, out1_hbm.at[pl.ds(base, _BPW)], sem1)
    cp2.wait()
    o2 = pltpu.async_copy(rows2_v, out2_hbm.at[pl.ds(base, _BPW)], sem2)
    o1.wait()
    o2.wait()


_BLK = 512


def _cls_body(x1_ref, x2_ref, dom_ref, ds_ref, wt_ref, b_ref,
              logits_ref, labels_ref):
    dom = dom_ref[:].reshape(_BLK, 1)
    acc = jnp.zeros((_BLK, _D), jnp.float32)
    for j in range(_NDOM):
        row = ds_ref[j, :]
        acc = jnp.where(dom == j, row[None, :], acc)
    x = jnp.concatenate([x1_ref[:] + acc, x2_ref[:] + acc], axis=-1)
    logits_t = lax.dot_general(wt_ref[:], x, (((1,), (1,)), ((), ())))
    logits_t = logits_t + b_ref[:][:, None]
    logits_ref[:] = logits_t
    m = jnp.max(logits_t, axis=0)
    idx = lax.broadcasted_iota(jnp.int32, (_L, _BLK), 0)
    labels_ref[:] = jnp.min(jnp.where(logits_t == m[None, :], idx, _L), axis=0)


_classify = pl.pallas_call(
    _cls_body,
    grid=(_B // _BLK,),
    in_specs=[
        pl.BlockSpec((_BLK, _D), lambda i: (i, 0)),
        pl.BlockSpec((_BLK, _D), lambda i: (i, 0)),
        pl.BlockSpec((_BLK,), lambda i: (i,)),
        pl.BlockSpec((_NDOM, _D), lambda i: (0, 0)),
        pl.BlockSpec((_L, 2 * _D), lambda i: (0, 0)),
        pl.BlockSpec((_L,), lambda i: (0,)),
    ],
    out_specs=[
        pl.BlockSpec((_L, _BLK), lambda i: (0, i)),
        pl.BlockSpec((_BLK,), lambda i: (i,)),
    ],
    out_shape=[
        jax.ShapeDtypeStruct((_L, _B), jnp.float32),
        jax.ShapeDtypeStruct((_B,), jnp.int32),
    ],
)


def kernel(sentences, entities_1, entities_2, domains, emb_table,
           dataset_embeds, W, b):
    sent_flat = sentences.astype(jnp.int32).reshape(_B * _S)
    e1 = entities_1.astype(jnp.int32)
    e2 = entities_2.astype(jnp.int32)
    dom = domains.astype(jnp.int32)
    x1, x2 = _sc_gather(sent_flat, e1, e2, emb_table)
    logits_t, labels_1d = _classify(x1, x2, dom, dataset_embeds, W.T, b)
    return labels_1d[:, None], logits_t.T


# speedup vs baseline: 1.0187x; 1.0112x over previous
import functools

import jax
import jax.numpy as jnp
from jax import lax
from jax.experimental import pallas as pl
from jax.experimental.pallas import tpu as pltpu
from jax.experimental.pallas import tpu_sc as plsc

_B, _S, _V, _D, _NDOM, _L = 1024, 50, 100000, 768, 6, 32

_NC, _NS, _LANES = 2, 16, 16
_NW = _NC * _NS
_BPW = _B // _NW

_mesh = plsc.VectorSubcoreMesh(core_axis_name="c", subcore_axis_name="s")


@functools.partial(
    pl.kernel,
    mesh=_mesh,
    out_type=[
        jax.ShapeDtypeStruct((_B, _D), jnp.float32),
        jax.ShapeDtypeStruct((_B, _D), jnp.float32),
    ],
    scratch_types=[
        pltpu.VMEM((_BPW,), jnp.int32),
        pltpu.VMEM((_BPW,), jnp.int32),
        pltpu.VMEM((_BPW,), jnp.int32),
        pltpu.VMEM((_BPW,), jnp.int32),
        pltpu.VMEM((_BPW,), jnp.int32),
        pltpu.VMEM((_BPW,), jnp.int32),
        pltpu.VMEM((_BPW, _D), jnp.float32),
        pltpu.VMEM((_BPW, _D), jnp.float32),
        pltpu.SemaphoreType.DMA,
        pltpu.SemaphoreType.DMA,
    ],
)
def _sc_gather(sent_hbm, e1_hbm, e2_hbm, table_hbm, out1_hbm, out2_hbm,
               e1_v, e2_v, flat1_v, flat2_v, tok1_v, tok2_v,
               rows1_v, rows2_v, sem1, sem2):
    wid = lax.axis_index("s") * _NC + lax.axis_index("c")
    base = wid * _BPW
    c1 = pltpu.async_copy(e1_hbm.at[pl.ds(base, _BPW)], e1_v, sem1)
    c2 = pltpu.async_copy(e2_hbm.at[pl.ds(base, _BPW)], e2_v, sem2)
    c1.wait()
    c2.wait()
    for k in range(_BPW // _LANES):
        sl = pl.ds(k * _LANES, _LANES)
        row_base = (lax.iota(jnp.int32, _LANES) + (base + k * _LANES)) * _S
        flat1_v[sl] = row_base + e1_v[sl]
        flat2_v[sl] = row_base + e2_v[sl]
    tk1 = pltpu.async_copy(sent_hbm.at[flat1_v], tok1_v, sem1)
    tk2 = pltpu.async_copy(sent_hbm.at[flat2_v], tok2_v, sem2)
    tk1.wait()
    cp1 = pltpu.async_copy(table_hbm.at[tok1_v], rows1_v, sem1)
    tk2.wait()
    cp2 = pltpu.async_copy(table_hbm.at[tok2_v], rows2_v, sem2)
    cp1.wait()
    o1 = pltpu.async_copy(rows1_v, out1_hbm.at[pl.ds(base, _BPW)], sem1)
    cp2.wait()
    o2 = pltpu.async_copy(rows2_v, out2_hbm.at[pl.ds(base, _BPW)], sem2)
    o1.wait()
    o2.wait()


_BLK = 512


def _cls_body(x1_ref, x2_ref, dom_ref, ds_ref, wt_ref, b_ref,
              logits_ref, labels_ref):
    dom = dom_ref[:].reshape(_BLK, 1)
    acc = jnp.zeros((_BLK, _D), jnp.float32)
    for j in range(_NDOM):
        row = ds_ref[j, :]
        acc = jnp.where(dom == j, row[None, :], acc)
    x = jnp.concatenate([x1_ref[:] + acc, x2_ref[:] + acc], axis=-1)
    logits_t = lax.dot_general(wt_ref[:], x, (((1,), (1,)), ((), ())))
    logits_t = logits_t + b_ref[:][:, None]
    logits_ref[:] = logits_t
    m = jnp.max(logits_t, axis=0)
    idx = lax.broadcasted_iota(jnp.int32, (_L, _BLK), 0)
    labels_ref[:] = jnp.min(jnp.where(logits_t == m[None, :], idx, _L), axis=0)


_classify = pl.pallas_call(
    _cls_body,
    grid=(_B // _BLK,),
    in_specs=[
        pl.BlockSpec((_BLK, _D), lambda i: (i, 0)),
        pl.BlockSpec((_BLK, _D), lambda i: (i, 0)),
        pl.BlockSpec((_BLK,), lambda i: (i,)),
        pl.BlockSpec((_NDOM, _D), lambda i: (0, 0)),
        pl.BlockSpec((_L, 2 * _D), lambda i: (0, 0)),
        pl.BlockSpec((_L,), lambda i: (0,)),
    ],
    out_specs=[
        pl.BlockSpec((_L, _BLK), lambda i: (0, i)),
        pl.BlockSpec((_BLK,), lambda i: (i,)),
    ],
    out_shape=[
        jax.ShapeDtypeStruct((_L, _B), jnp.float32),
        jax.ShapeDtypeStruct((_B,), jnp.int32),
    ],
)


def kernel(sentences, entities_1, entities_2, domains, emb_table,
           dataset_embeds, W, b):
    sent_flat = sentences.astype(jnp.int32).reshape(_B * _S)
    e1 = entities_1.astype(jnp.int32)
    e2 = entities_2.astype(jnp.int32)
    dom = domains.astype(jnp.int32)
    x1, x2 = _sc_gather(sent_flat, e1, e2, emb_table)
    logits_t, labels_1d = _classify(x1, x2, dom, dataset_embeds, W.T, b)
    return labels_1d[:, None], logits_t.T


# speedup vs baseline: 1.0234x; 1.0046x over previous
import functools

import jax
import jax.numpy as jnp
from jax import lax
from jax.experimental import pallas as pl
from jax.experimental.pallas import tpu as pltpu
from jax.experimental.pallas import tpu_sc as plsc

_B, _S, _V, _D, _NDOM, _L = 1024, 50, 100000, 768, 6, 32

_NC, _NS, _LANES = 2, 16, 16
_NW = _NC * _NS
_BPW = _B // _NW

_mesh = plsc.VectorSubcoreMesh(core_axis_name="c", subcore_axis_name="s")


@functools.partial(
    pl.kernel,
    mesh=_mesh,
    out_type=[
        jax.ShapeDtypeStruct((_B, _D), jnp.float32),
        jax.ShapeDtypeStruct((_B, _D), jnp.float32),
    ],
    scratch_types=[
        pltpu.VMEM((_BPW,), jnp.int32),
        pltpu.VMEM((_BPW,), jnp.int32),
        pltpu.VMEM((_BPW,), jnp.int32),
        pltpu.VMEM((_BPW,), jnp.int32),
        pltpu.VMEM((_BPW,), jnp.int32),
        pltpu.VMEM((_BPW,), jnp.int32),
        pltpu.VMEM((_BPW, _D), jnp.float32),
        pltpu.VMEM((_BPW, _D), jnp.float32),
        pltpu.SemaphoreType.DMA,
        pltpu.SemaphoreType.DMA,
    ],
)
def _sc_gather(sent_hbm, e1_hbm, e2_hbm, table_hbm, out1_hbm, out2_hbm,
               e1_v, e2_v, flat1_v, flat2_v, tok1_v, tok2_v,
               rows1_v, rows2_v, sem1, sem2):
    wid = lax.axis_index("s") * _NC + lax.axis_index("c")
    base = wid * _BPW
    c1 = pltpu.async_copy(e1_hbm.at[pl.ds(base, _BPW)], e1_v, sem1)
    c2 = pltpu.async_copy(e2_hbm.at[pl.ds(base, _BPW)], e2_v, sem2)
    c1.wait()
    for k in range(_BPW // _LANES):
        sl = pl.ds(k * _LANES, _LANES)
        row_base = (lax.iota(jnp.int32, _LANES) + (base + k * _LANES)) * _S
        flat1_v[sl] = row_base + e1_v[sl]
    tk1 = pltpu.async_copy(sent_hbm.at[flat1_v], tok1_v, sem1)
    c2.wait()
    for k in range(_BPW // _LANES):
        sl = pl.ds(k * _LANES, _LANES)
        row_base = (lax.iota(jnp.int32, _LANES) + (base + k * _LANES)) * _S
        flat2_v[sl] = row_base + e2_v[sl]
    tk2 = pltpu.async_copy(sent_hbm.at[flat2_v], tok2_v, sem2)
    tk1.wait()
    cp1 = pltpu.async_copy(table_hbm.at[tok1_v], rows1_v, sem1)
    tk2.wait()
    cp2 = pltpu.async_copy(table_hbm.at[tok2_v], rows2_v, sem2)
    cp1.wait()
    o1 = pltpu.async_copy(rows1_v, out1_hbm.at[pl.ds(base, _BPW)], sem1)
    cp2.wait()
    o2 = pltpu.async_copy(rows2_v, out2_hbm.at[pl.ds(base, _BPW)], sem2)
    o1.wait()
    o2.wait()


_BLK = 512


def _cls_body(x1_ref, x2_ref, dom_ref, ds_ref, wt_ref, b_ref,
              logits_ref, labels_ref):
    dom = dom_ref[:].reshape(_BLK, 1)
    acc = jnp.zeros((_BLK, _D), jnp.float32)
    for j in range(_NDOM):
        row = ds_ref[j, :]
        acc = jnp.where(dom == j, row[None, :], acc)
    x = jnp.concatenate([x1_ref[:] + acc, x2_ref[:] + acc], axis=-1)
    logits_t = lax.dot_general(wt_ref[:], x, (((1,), (1,)), ((), ())))
    logits_t = logits_t + b_ref[:][:, None]
    logits_ref[:] = logits_t
    m = jnp.max(logits_t, axis=0)
    idx = lax.broadcasted_iota(jnp.int32, (_L, _BLK), 0)
    labels_ref[:] = jnp.min(jnp.where(logits_t == m[None, :], idx, _L), axis=0)


_classify = pl.pallas_call(
    _cls_body,
    grid=(_B // _BLK,),
    in_specs=[
        pl.BlockSpec((_BLK, _D), lambda i: (i, 0)),
        pl.BlockSpec((_BLK, _D), lambda i: (i, 0)),
        pl.BlockSpec((_BLK,), lambda i: (i,)),
        pl.BlockSpec((_NDOM, _D), lambda i: (0, 0)),
        pl.BlockSpec((_L, 2 * _D), lambda i: (0, 0)),
        pl.BlockSpec((_L,), lambda i: (0,)),
    ],
    out_specs=[
        pl.BlockSpec((_L, _BLK), lambda i: (0, i)),
        pl.BlockSpec((_BLK,), lambda i: (i,)),
    ],
    out_shape=[
        jax.ShapeDtypeStruct((_L, _B), jnp.float32),
        jax.ShapeDtypeStruct((_B,), jnp.int32),
    ],
)


def kernel(sentences, entities_1, entities_2, domains, emb_table,
           dataset_embeds, W, b):
    sent_flat = sentences.astype(jnp.int32).reshape(_B * _S)
    e1 = entities_1.astype(jnp.int32)
    e2 = entities_2.astype(jnp.int32)
    dom = domains.astype(jnp.int32)
    x1, x2 = _sc_gather(sent_flat, e1, e2, emb_table)
    logits_t, labels_1d = _classify(x1, x2, dom, dataset_embeds, W.T, b)
    return labels_1d[:, None], logits_t.T
